# Initial kernel scaffold; baseline (speedup 1.0000x reference)
import jax
import jax.numpy as jnp
from jax.experimental import pallas as pl


def kernel(bilateral_grid, guide, input):
    raise NotImplementedError("write your pallas kernel here")


# speedup vs baseline: 1130.4880x; 1130.4880x over previous
import functools

import jax
import jax.numpy as jnp
from jax.experimental import pallas as pl
from jax.experimental.pallas import tpu as pltpu

_C = 12
_GD = 8
_GH = 16
_GW = 16
_CI = 3
_CO = 3
_NIN = 4
_TH = 16


def _slice_body(grid_ref, guide_ref, inp_ref, out_ref, gx_s, *, H, W):
    t = pl.program_id(1)
    sx = W // _GW
    sy = H // _GH

    @pl.when(t == 0)
    def _():
        xi = jax.lax.broadcasted_iota(jnp.int32, (_GW, W), 0).astype(
            jnp.float32
        )
        wi = jax.lax.broadcasted_iota(jnp.int32, (_GW, W), 1).astype(
            jnp.float32
        )
        ux = jnp.clip((wi + 0.5) / sx - 0.5, 0.0, _GW - 1.0)
        mxT = jnp.maximum(1.0 - jnp.abs(xi - ux), 0.0)
        g = grid_ref[0].reshape(_GH * _C * _GD, _GW)
        gx_s[...] = jnp.dot(
            g, mxT, preferred_element_type=jnp.float32
        ).reshape(_GH, _C * _GD, W)

    hh = (
        jax.lax.broadcasted_iota(jnp.int32, (_TH, 1), 0).astype(jnp.float32)
        + t * _TH
    )
    uy = jnp.clip((hh + 0.5) / sy - 0.5, 0.0, _GH - 1.0)
    S = jnp.clip((t - 1) // 2, 0, _GH - 2)
    w1 = jnp.clip(uy - S.astype(jnp.float32), 0.0, 1.0)
    w0 = 1.0 - w1

    GA = gx_s[S]
    GB = gx_s[S + 1]

    uz = jnp.clip(guide_ref[0] * _GD - 0.5, 0.0, _GD - 1.0)
    WA = []
    WB = []
    for k in range(_GD):
        wk = jnp.maximum(1.0 - jnp.abs(uz - float(k)), 0.0)
        WA.append(wk * w0)
        WB.append(wk * w1)

    coeff = []
    for j in range(_C):
        acc = WA[0] * GA[j * _GD]
        acc += WB[0] * GB[j * _GD]
        for k in range(1, _GD):
            acc += WA[k] * GA[j * _GD + k]
            acc += WB[k] * GB[j * _GD + k]
        coeff.append(acc)

    for c in range(_CO):
        oc = coeff[c * _NIN + _CI]
        for i in range(_CI):
            oc += coeff[c * _NIN + i] * inp_ref[0, i]
        out_ref[0, c] = oc


def kernel(bilateral_grid, guide, input):
    N, C, gd, gh, gw = bilateral_grid.shape
    _, ci, H, W = input.shape
    grid_r = jnp.transpose(bilateral_grid, (0, 3, 1, 2, 4)).reshape(
        N, gh, C * gd, gw
    )
    nt = H // _TH
    body = functools.partial(_slice_body, H=H, W=W)
    return pl.pallas_call(
        body,
        grid=(N, nt),
        in_specs=[
            pl.BlockSpec((1, gh, C * gd, gw), lambda n, t: (n, 0, 0, 0)),
            pl.BlockSpec((1, _TH, W), lambda n, t: (n, t, 0)),
            pl.BlockSpec((1, ci, _TH, W), lambda n, t: (n, 0, t, 0)),
        ],
        out_specs=pl.BlockSpec((1, _CO, _TH, W), lambda n, t: (n, 0, t, 0)),
        out_shape=jax.ShapeDtypeStruct((N, _CO, H, W), jnp.float32),
        scratch_shapes=[pltpu.VMEM((gh, C * gd, W), jnp.float32)],
    )(grid_r, guide, input)
